# baseline (speedup 1.0000x reference)
import functools

import jax
import jax.numpy as jnp
from jax import lax
from jax.experimental import pallas as pl
from jax.experimental.pallas import tpu as pltpu
from jax.experimental.pallas import tpu_sc as plsc

_N_NODES = 10000
_N_EDGES = 320000
_D = 128
_BN_EPS = 1e-3

_NC = 2
_NS = 16
_NTILES = _NC * _NS
_E_CHK = 128
_CHK_PER_TILE = 80
_E_PER_TILE = _E_CHK * _CHK_PER_TILE
_E_PAD = _NTILES * _E_PER_TILE
_N_CHK_TOT = _E_PAD // _E_CHK
_ROWS_PER_TILE = 624
_ROWS_REM = _N_NODES - _NS * _ROWS_PER_TILE
_ZBUF = 16


def _sc_body(h_hbm, row_hbm, col_hbm, val_hbm, out_hbm,
             row_v0, row_v1, col_v0, col_v1, val_v0, val_v1,
             rows_a, rows_b, zeros_v, acc_sh,
             sem_a, sem_b, semi_a, semi_b):
    c = lax.axis_index("c")
    s = lax.axis_index("s")
    tile = c * _NS + s

    def _zrow(i, _):
        for j in range(_D // 16):
            zeros_v[i, pl.ds(j * 16, 16)] = jnp.zeros((16,), jnp.float32)
        return 0
    lax.fori_loop(0, _ZBUF, _zrow, 0)
    for k in range(_ROWS_PER_TILE // _ZBUF):
        pltpu.sync_copy(zeros_v, acc_sh.at[pl.ds(s * _ROWS_PER_TILE + k * _ZBUF, _ZBUF)])

    @pl.when(s == 0)
    def _zero_rem():
        pltpu.sync_copy(zeros_v.at[pl.ds(0, _ROWS_REM)],
                        acc_sh.at[pl.ds(_NS * _ROWS_PER_TILE, _ROWS_REM)])
    plsc.subcore_barrier()

    e_base = tile * _E_PER_TILE
    bufs = ((row_v0, col_v0, val_v0, rows_a, sem_a, semi_a),
            (row_v1, col_v1, val_v1, rows_b, sem_b, semi_b))

    def _fetch_idx(b, g):
        rv, cv, vv, _, _, si = bufs[b]
        base = e_base + g * _E_CHK
        pltpu.async_copy(row_hbm.at[pl.ds(base, _E_CHK)], rv, si)
        pltpu.async_copy(col_hbm.at[pl.ds(base, _E_CHK)], cv, si)
        pltpu.async_copy(val_hbm.at[pl.ds(base, _E_CHK)], vv, si)

    def _issue_gather(b):
        rv, cv, vv, rows, sg, si = bufs[b]
        pltpu.make_async_copy(row_hbm.at[pl.ds(0, _E_CHK)], rv, si).wait()
        pltpu.make_async_copy(row_hbm.at[pl.ds(0, _E_CHK)], cv, si).wait()
        pltpu.make_async_copy(val_hbm.at[pl.ds(0, _E_CHK)], vv, si).wait()
        pltpu.async_copy(h_hbm.at[cv], rows, sg)

    def _drain(b):
        rv, cv, vv, rows, sg, si = bufs[b]
        pltpu.make_async_copy(h_hbm.at[cv], rows, sg).wait()

        def _edge(e, _):
            vb = plsc.load_gather(vv, [jnp.full((16,), e, jnp.int32)])
            for j in range(_D // 16):
                sl = pl.ds(j * 16, 16)
                rows[e, sl] = rows[e, sl] * vb
            return 0
        lax.fori_loop(0, _E_CHK, _edge, 0)
        pltpu.sync_copy(rows, acc_sh.at[rv], add=True)

    _fetch_idx(0, 0)
    _fetch_idx(1, 1)
    _issue_gather(0)

    def _pair(p, _):
        g0 = 2 * p
        more = p < _CHK_PER_TILE // 2 - 1
        _issue_gather(1)
        _drain(0)

        @pl.when(more)
        def _pf0():
            _fetch_idx(0, g0 + 2)
        _drain(1)

        @pl.when(more)
        def _pf1():
            _fetch_idx(1, g0 + 3)
            _issue_gather(0)
        return 0
    lax.fori_loop(0, _CHK_PER_TILE // 2, _pair, 0)

    plsc.subcore_barrier()

    pltpu.sync_copy(acc_sh.at[pl.ds(s * _ROWS_PER_TILE, _ROWS_PER_TILE)],
                    out_hbm.at[c, pl.ds(s * _ROWS_PER_TILE, _ROWS_PER_TILE)])

    @pl.when(s == 0)
    def _write_rem():
        pltpu.sync_copy(acc_sh.at[pl.ds(_NS * _ROWS_PER_TILE, _ROWS_REM)],
                        out_hbm.at[c, pl.ds(_NS * _ROWS_PER_TILE, _ROWS_REM)])


@jax.jit
def _sc_segment_sum(H, row, col, val):
    mesh = plsc.VectorSubcoreMesh(core_axis_name="c", subcore_axis_name="s")
    return pl.kernel(
        _sc_body,
        out_type=jax.ShapeDtypeStruct((_NC, _N_NODES, _D), jnp.float32),
        mesh=mesh,
        compiler_params=pltpu.CompilerParams(needs_layout_passes=False),
        scratch_types=[
            pltpu.VMEM((_E_CHK,), jnp.int32),
            pltpu.VMEM((_E_CHK,), jnp.int32),
            pltpu.VMEM((_E_CHK,), jnp.int32),
            pltpu.VMEM((_E_CHK,), jnp.int32),
            pltpu.VMEM((_E_CHK,), jnp.float32),
            pltpu.VMEM((_E_CHK,), jnp.float32),
            pltpu.VMEM((_E_CHK, _D), jnp.float32),
            pltpu.VMEM((_E_CHK, _D), jnp.float32),
            pltpu.VMEM((_ZBUF, _D), jnp.float32),
            pltpu.VMEM_SHARED((_N_NODES, _D), jnp.float32),
            pltpu.SemaphoreType.DMA,
            pltpu.SemaphoreType.DMA,
            pltpu.SemaphoreType.DMA,
            pltpu.SemaphoreType.DMA,
        ],
    )(H, row, col, val)


def _tc_body(h0_ref, h1_ref, scale_ref, shift_ref, w_ref, b_ref, o_ref):
    x = h0_ref[...] + h1_ref[...]
    x = x * scale_ref[...] + shift_ref[...]
    y = jnp.dot(x, w_ref[...], preferred_element_type=jnp.float32) + b_ref[...]
    o_ref[...] = 0.5 * y * (1.0 + lax.erf(y * 0.7071067811865476))


@jax.jit
def _tc_ffn(h0, h1, scale, shift, W, b):
    blk = 1000
    grid = (_N_NODES // blk,)
    return pl.pallas_call(
        _tc_body,
        grid=grid,
        in_specs=[
            pl.BlockSpec((blk, _D), lambda i: (i, 0)),
            pl.BlockSpec((blk, _D), lambda i: (i, 0)),
            pl.BlockSpec((1, _D), lambda i: (0, 0)),
            pl.BlockSpec((1, _D), lambda i: (0, 0)),
            pl.BlockSpec((_D, _D), lambda i: (0, 0)),
            pl.BlockSpec((1, _D), lambda i: (0, 0)),
        ],
        out_specs=pl.BlockSpec((blk, _D), lambda i: (i, 0)),
        out_shape=jax.ShapeDtypeStruct((_N_NODES, _D), jnp.float32),
    )(h0, h1, scale, shift, W, b)


def kernel(H, edge_index, adj_values, gamma, beta, moving_mean, moving_var, W, b):
    npad = _E_PAD - _N_EDGES
    row = jnp.pad(edge_index[0], (0, npad))
    col = jnp.pad(edge_index[1], (0, npad))
    val = jnp.pad(adj_values, (0, npad))
    hpart = _sc_segment_sum(H, row, col, val)
    scale = gamma * lax.rsqrt(moving_var + _BN_EPS)
    shift = beta - moving_mean * scale
    return _tc_ffn(hpart[0], hpart[1], scale.reshape(1, _D),
                   shift.reshape(1, _D), W, b.reshape(1, _D))


# speedup vs baseline: 1.3033x; 1.3033x over previous
import functools

import jax
import jax.numpy as jnp
from jax import lax
from jax.experimental import pallas as pl
from jax.experimental.pallas import tpu as pltpu
from jax.experimental.pallas import tpu_sc as plsc

_N_NODES = 10000
_N_EDGES = 320000
_D = 128
_BN_EPS = 1e-3

_NC = 2
_NS = 16
_NTILES = _NC * _NS
_E_CHK = 128
_CHK_PER_TILE = 80
_E_PER_TILE = _E_CHK * _CHK_PER_TILE
_E_PAD = _NTILES * _E_PER_TILE
_N_CHK_TOT = _E_PAD // _E_CHK
_ROWS_PER_TILE = 624
_ROWS_REM = _N_NODES - _NS * _ROWS_PER_TILE
_ZBUF = 16


def _sc_body(h_hbm, row_hbm, col_hbm, val_hbm, out_hbm,
             row_v0, row_v1, col_v0, col_v1, val_v0, val_v1,
             rows_a, rows_b, rowsf, zeros_v, acc_sh,
             sem_a, sem_b, semi_a, semi_b):
    c = lax.axis_index("c")
    s = lax.axis_index("s")
    tile = c * _NS + s

    def _zrow(i, _):
        for j in range(_D // 16):
            zeros_v[i, pl.ds(j * 16, 16)] = jnp.zeros((16,), jnp.float32)
        return 0
    lax.fori_loop(0, _ZBUF, _zrow, 0)
    for k in range(_ROWS_PER_TILE // _ZBUF):
        pltpu.sync_copy(zeros_v, acc_sh.at[pl.ds(s * _ROWS_PER_TILE + k * _ZBUF, _ZBUF)])

    @pl.when(s == 0)
    def _zero_rem():
        pltpu.sync_copy(zeros_v.at[pl.ds(0, _ROWS_REM)],
                        acc_sh.at[pl.ds(_NS * _ROWS_PER_TILE, _ROWS_REM)])
    plsc.subcore_barrier()

    e_base = tile * _E_PER_TILE
    bufs = ((row_v0, col_v0, val_v0, rows_a, sem_a, semi_a),
            (row_v1, col_v1, val_v1, rows_b, sem_b, semi_b))

    def _fetch_idx(b, g):
        rv, cv, vv, _, _, si = bufs[b]
        base = e_base + g * _E_CHK
        pltpu.async_copy(row_hbm.at[pl.ds(base, _E_CHK)], rv, si)
        pltpu.async_copy(col_hbm.at[pl.ds(base, _E_CHK)], cv, si)
        pltpu.async_copy(val_hbm.at[pl.ds(base, _E_CHK)], vv, si)

    def _issue_gather(b):
        rv, cv, vv, rows, sg, si = bufs[b]
        pltpu.make_async_copy(row_hbm.at[pl.ds(0, _E_CHK)], rv, si).wait()
        pltpu.make_async_copy(row_hbm.at[pl.ds(0, _E_CHK)], cv, si).wait()
        pltpu.make_async_copy(val_hbm.at[pl.ds(0, _E_CHK)], vv, si).wait()
        pltpu.async_copy(h_hbm.at[cv], rows, sg)

    def _drain(b):
        rv, cv, vv, rows, sg, si = bufs[b]
        pltpu.make_async_copy(h_hbm.at[cv], rows, sg).wait()

        def _edge(e, _):
            vb = plsc.load_gather(vv, [jnp.full((16,), e, jnp.int32)])
            for j in range(_D // 32):
                pair = plsc.bitcast(rows[e, pl.ds(j * 16, 16)], jnp.bfloat16)
                lo, hi = plsc.unpack(pair, format=plsc.PackFormat.INTERLEAVED)
                rowsf[e, pl.ds(j * 32, 16)] = lo * vb
                rowsf[e, pl.ds(j * 32 + 16, 16)] = hi * vb
            return 0
        lax.fori_loop(0, _E_CHK, _edge, 0)
        pltpu.sync_copy(rowsf, acc_sh.at[rv], add=True)

    _fetch_idx(0, 0)
    _fetch_idx(1, 1)
    _issue_gather(0)

    def _pair(p, _):
        g0 = 2 * p
        more = p < _CHK_PER_TILE // 2 - 1
        _issue_gather(1)
        _drain(0)

        @pl.when(more)
        def _pf0():
            _fetch_idx(0, g0 + 2)
        _drain(1)

        @pl.when(more)
        def _pf1():
            _fetch_idx(1, g0 + 3)
            _issue_gather(0)
        return 0
    lax.fori_loop(0, _CHK_PER_TILE // 2, _pair, 0)

    plsc.subcore_barrier()

    pltpu.sync_copy(acc_sh.at[pl.ds(s * _ROWS_PER_TILE, _ROWS_PER_TILE)],
                    out_hbm.at[c, pl.ds(s * _ROWS_PER_TILE, _ROWS_PER_TILE)])

    @pl.when(s == 0)
    def _write_rem():
        pltpu.sync_copy(acc_sh.at[pl.ds(_NS * _ROWS_PER_TILE, _ROWS_REM)],
                        out_hbm.at[c, pl.ds(_NS * _ROWS_PER_TILE, _ROWS_REM)])


@jax.jit
def _sc_segment_sum(H, row, col, val):
    mesh = plsc.VectorSubcoreMesh(core_axis_name="c", subcore_axis_name="s")
    return pl.kernel(
        _sc_body,
        out_type=jax.ShapeDtypeStruct((_NC, _N_NODES, _D), jnp.float32),
        mesh=mesh,
        compiler_params=pltpu.CompilerParams(needs_layout_passes=False,
                                             use_tc_tiling_on_sc=False),
        scratch_types=[
            pltpu.VMEM((_E_CHK,), jnp.int32),
            pltpu.VMEM((_E_CHK,), jnp.int32),
            pltpu.VMEM((_E_CHK,), jnp.int32),
            pltpu.VMEM((_E_CHK,), jnp.int32),
            pltpu.VMEM((_E_CHK,), jnp.float32),
            pltpu.VMEM((_E_CHK,), jnp.float32),
            pltpu.VMEM((_E_CHK, _D // 2), jnp.int32),
            pltpu.VMEM((_E_CHK, _D // 2), jnp.int32),
            pltpu.VMEM((_E_CHK, _D), jnp.float32),
            pltpu.VMEM((_ZBUF, _D), jnp.float32),
            pltpu.VMEM_SHARED((_N_NODES, _D), jnp.float32),
            pltpu.SemaphoreType.DMA,
            pltpu.SemaphoreType.DMA,
            pltpu.SemaphoreType.DMA,
            pltpu.SemaphoreType.DMA,
        ],
    )(H, row, col, val)


def _tc_body(h0_ref, h1_ref, scale_ref, shift_ref, w_ref, b_ref, o_ref):
    x = h0_ref[...] + h1_ref[...]
    x = x * scale_ref[...] + shift_ref[...]
    y = jnp.dot(x, w_ref[...], preferred_element_type=jnp.float32) + b_ref[...]
    o_ref[...] = 0.5 * y * (1.0 + lax.erf(y * 0.7071067811865476))


@jax.jit
def _tc_ffn(h0, h1, scale, shift, W, b):
    blk = 1000
    grid = (_N_NODES // blk,)
    return pl.pallas_call(
        _tc_body,
        grid=grid,
        in_specs=[
            pl.BlockSpec((blk, _D), lambda i: (i, 0)),
            pl.BlockSpec((blk, _D), lambda i: (i, 0)),
            pl.BlockSpec((1, _D), lambda i: (0, 0)),
            pl.BlockSpec((1, _D), lambda i: (0, 0)),
            pl.BlockSpec((_D, _D), lambda i: (0, 0)),
            pl.BlockSpec((1, _D), lambda i: (0, 0)),
        ],
        out_specs=pl.BlockSpec((blk, _D), lambda i: (i, 0)),
        out_shape=jax.ShapeDtypeStruct((_N_NODES, _D), jnp.float32),
    )(h0, h1, scale, shift, W, b)


def kernel(H, edge_index, adj_values, gamma, beta, moving_mean, moving_var, W, b):
    npad = _E_PAD - _N_EDGES
    row = jnp.pad(edge_index[0], (0, npad))
    col = jnp.pad(edge_index[1], (0, npad))
    val = jnp.pad(adj_values, (0, npad))
    perm = jnp.arange(_D).reshape(4, 2, 16).transpose(0, 2, 1).reshape(_D)
    hb = H[:, perm].astype(jnp.bfloat16).reshape(_N_NODES, _D // 2, 2)
    hpacked = lax.bitcast_convert_type(hb, jnp.int32)
    hpart = _sc_segment_sum(hpacked, row, col, val)
    scale = gamma * lax.rsqrt(moving_var + _BN_EPS)
    shift = beta - moving_mean * scale
    return _tc_ffn(hpart[0], hpart[1], scale.reshape(1, _D),
                   shift.reshape(1, _D), W, b.reshape(1, _D))


# speedup vs baseline: 1.3665x; 1.0485x over previous
import functools

import jax
import jax.numpy as jnp
from jax import lax
from jax.experimental import pallas as pl
from jax.experimental.pallas import tpu as pltpu
from jax.experimental.pallas import tpu_sc as plsc

_N_NODES = 10000
_N_EDGES = 320000
_D = 128
_BN_EPS = 1e-3

_NC = 2
_NS = 16
_NTILES = _NC * _NS
_E_CHK = 128
_CHK_PER_TILE = 80
_E_PER_TILE = _E_CHK * _CHK_PER_TILE
_E_PAD = _NTILES * _E_PER_TILE
_N_CHK_TOT = _E_PAD // _E_CHK
_ROWS_PER_TILE = 624
_ROWS_REM = _N_NODES - _NS * _ROWS_PER_TILE
_ZBUF = 8


def _sc_body(h_hbm, row_hbm, col_hbm, val_hbm, out_hbm,
             row_v0, row_v1, col_v0, col_v1, val_v0, val_v1,
             rows_a, rows_b, rowsf_a, rowsf_b, zeros_v, acc_sh,
             sem_a, sem_b, semi_a, semi_b, sems_a, sems_b):
    c = lax.axis_index("c")
    s = lax.axis_index("s")
    tile = c * _NS + s

    def _zrow(i, _):
        for j in range(_D // 16):
            zeros_v[i, pl.ds(j * 16, 16)] = jnp.zeros((16,), jnp.float32)
        return 0
    lax.fori_loop(0, _ZBUF, _zrow, 0)
    for k in range(_ROWS_PER_TILE // _ZBUF):
        pltpu.sync_copy(zeros_v, acc_sh.at[pl.ds(s * _ROWS_PER_TILE + k * _ZBUF, _ZBUF)])

    @pl.when(s == 0)
    def _zero_rem():
        pltpu.sync_copy(zeros_v.at[pl.ds(0, _ROWS_REM)],
                        acc_sh.at[pl.ds(_NS * _ROWS_PER_TILE, _ROWS_REM)])
    plsc.subcore_barrier()

    e_base = tile * _E_PER_TILE
    bufs = ((row_v0, col_v0, val_v0, rows_a, rowsf_a, sem_a, semi_a, sems_a),
            (row_v1, col_v1, val_v1, rows_b, rowsf_b, sem_b, semi_b, sems_b))

    def _fetch_idx(b, g):
        rv, cv, vv, _, _, _, si, _ = bufs[b]
        base = e_base + g * _E_CHK
        pltpu.async_copy(row_hbm.at[pl.ds(base, _E_CHK)], rv, si)
        pltpu.async_copy(col_hbm.at[pl.ds(base, _E_CHK)], cv, si)
        pltpu.async_copy(val_hbm.at[pl.ds(base, _E_CHK)], vv, si)

    def _issue_gather(b):
        rv, cv, vv, rows, _, sg, si, _ = bufs[b]
        pltpu.make_async_copy(row_hbm.at[pl.ds(0, _E_CHK)], rv, si).wait()
        pltpu.make_async_copy(row_hbm.at[pl.ds(0, _E_CHK)], cv, si).wait()
        pltpu.make_async_copy(val_hbm.at[pl.ds(0, _E_CHK)], vv, si).wait()
        pltpu.async_copy(h_hbm.at[cv], rows, sg)

    def _wait_scatter(b):
        rv, cv, vv, rows, rf, sg, si, ss = bufs[b]
        pltpu.make_async_copy(rf, acc_sh.at[rv], ss).wait()

    def _drain(b):
        rv, cv, vv, rows, rf, sg, si, ss = bufs[b]
        pltpu.make_async_copy(h_hbm.at[cv], rows, sg).wait()

        def _edge(e, _):
            vb = plsc.load_gather(vv, [jnp.full((16,), e, jnp.int32)])
            for j in range(_D // 32):
                w = rows[e, pl.ds(j * 16, 16)]
                lo = plsc.bitcast(w << 16, jnp.float32)
                hi = plsc.bitcast(w & jnp.int32(-65536), jnp.float32)
                rf[e, pl.ds(j * 32, 16)] = lo * vb
                rf[e, pl.ds(j * 32 + 16, 16)] = hi * vb
            return 0
        lax.fori_loop(0, _E_CHK, _edge, 0)
        pltpu.async_copy(rf, acc_sh.at[rv], ss, add=True)

    _fetch_idx(0, 0)
    _fetch_idx(1, 1)
    _issue_gather(0)

    def _pair(p, _):
        g0 = 2 * p
        more = p < _CHK_PER_TILE // 2 - 1
        _issue_gather(1)

        @pl.when(p > 0)
        def _ws0():
            _wait_scatter(0)
        _drain(0)

        @pl.when(more)
        def _pf0():
            _fetch_idx(0, g0 + 2)

        @pl.when(p > 0)
        def _ws1():
            _wait_scatter(1)
        _drain(1)

        @pl.when(more)
        def _pf1():
            _fetch_idx(1, g0 + 3)
            _issue_gather(0)
        return 0
    lax.fori_loop(0, _CHK_PER_TILE // 2, _pair, 0)
    _wait_scatter(0)
    _wait_scatter(1)

    plsc.subcore_barrier()

    pltpu.sync_copy(acc_sh.at[pl.ds(s * _ROWS_PER_TILE, _ROWS_PER_TILE)],
                    out_hbm.at[c, pl.ds(s * _ROWS_PER_TILE, _ROWS_PER_TILE)])

    @pl.when(s == 0)
    def _write_rem():
        pltpu.sync_copy(acc_sh.at[pl.ds(_NS * _ROWS_PER_TILE, _ROWS_REM)],
                        out_hbm.at[c, pl.ds(_NS * _ROWS_PER_TILE, _ROWS_REM)])


@jax.jit
def _sc_segment_sum(H, row, col, val):
    mesh = plsc.VectorSubcoreMesh(core_axis_name="c", subcore_axis_name="s")
    return pl.kernel(
        _sc_body,
        out_type=jax.ShapeDtypeStruct((_NC, _N_NODES, _D), jnp.float32),
        mesh=mesh,
        compiler_params=pltpu.CompilerParams(needs_layout_passes=False,
                                             use_tc_tiling_on_sc=False),
        scratch_types=[
            pltpu.VMEM((_E_CHK,), jnp.int32),
            pltpu.VMEM((_E_CHK,), jnp.int32),
            pltpu.VMEM((_E_CHK,), jnp.int32),
            pltpu.VMEM((_E_CHK,), jnp.int32),
            pltpu.VMEM((_E_CHK,), jnp.float32),
            pltpu.VMEM((_E_CHK,), jnp.float32),
            pltpu.VMEM((_E_CHK, _D // 2), jnp.int32),
            pltpu.VMEM((_E_CHK, _D // 2), jnp.int32),
            pltpu.VMEM((_E_CHK, _D), jnp.float32),
            pltpu.VMEM((_E_CHK, _D), jnp.float32),
            pltpu.VMEM((_ZBUF, _D), jnp.float32),
            pltpu.VMEM_SHARED((_N_NODES, _D), jnp.float32),
            pltpu.SemaphoreType.DMA,
            pltpu.SemaphoreType.DMA,
            pltpu.SemaphoreType.DMA,
            pltpu.SemaphoreType.DMA,
            pltpu.SemaphoreType.DMA,
            pltpu.SemaphoreType.DMA,
        ],
    )(H, row, col, val)


def _tc_body(h0_ref, h1_ref, scale_ref, shift_ref, w_ref, b_ref, o_ref):
    x = h0_ref[...] + h1_ref[...]
    x = x * scale_ref[...] + shift_ref[...]
    y = jnp.dot(x, w_ref[...], preferred_element_type=jnp.float32) + b_ref[...]
    o_ref[...] = 0.5 * y * (1.0 + lax.erf(y * 0.7071067811865476))


@jax.jit
def _tc_ffn(h0, h1, scale, shift, W, b):
    blk = 1000
    grid = (_N_NODES // blk,)
    return pl.pallas_call(
        _tc_body,
        grid=grid,
        in_specs=[
            pl.BlockSpec((blk, _D), lambda i: (i, 0)),
            pl.BlockSpec((blk, _D), lambda i: (i, 0)),
            pl.BlockSpec((1, _D), lambda i: (0, 0)),
            pl.BlockSpec((1, _D), lambda i: (0, 0)),
            pl.BlockSpec((_D, _D), lambda i: (0, 0)),
            pl.BlockSpec((1, _D), lambda i: (0, 0)),
        ],
        out_specs=pl.BlockSpec((blk, _D), lambda i: (i, 0)),
        out_shape=jax.ShapeDtypeStruct((_N_NODES, _D), jnp.float32),
    )(h0, h1, scale, shift, W, b)


def kernel(H, edge_index, adj_values, gamma, beta, moving_mean, moving_var, W, b):
    npad = _E_PAD - _N_EDGES
    row = jnp.pad(edge_index[0], (0, npad))
    col = jnp.pad(edge_index[1], (0, npad))
    val = jnp.pad(adj_values, (0, npad))
    perm = jnp.arange(_D).reshape(4, 2, 16).transpose(0, 2, 1).reshape(_D)
    hb = H[:, perm].astype(jnp.bfloat16).reshape(_N_NODES, _D // 2, 2)
    hpacked = lax.bitcast_convert_type(hb, jnp.int32)
    hpart = _sc_segment_sum(hpacked, row, col, val)
    scale = gamma * lax.rsqrt(moving_var + _BN_EPS)
    shift = beta - moving_mean * scale
    return _tc_ffn(hpart[0], hpart[1], scale.reshape(1, _D),
                   shift.reshape(1, _D), W, b.reshape(1, _D))


# speedup vs baseline: 1.3854x; 1.0139x over previous
import functools

import jax
import jax.numpy as jnp
from jax import lax
from jax.experimental import pallas as pl
from jax.experimental.pallas import tpu as pltpu
from jax.experimental.pallas import tpu_sc as plsc

_N_NODES = 10000
_N_EDGES = 320000
_D = 128
_BN_EPS = 1e-3

_NC = 2
_NS = 16
_NTILES = _NC * _NS
_E_CHK = 128
_CHK_PER_TILE = 80
_E_PER_TILE = _E_CHK * _CHK_PER_TILE
_E_PAD = _NTILES * _E_PER_TILE
_N_CHK_TOT = _E_PAD // _E_CHK
_ROWS_PER_TILE = 624
_ROWS_REM = _N_NODES - _NS * _ROWS_PER_TILE
_ZBUF = 8


def _sc_body(h_hbm, row_hbm, col_hbm, val_hbm, out_hbm,
             row_v0, row_v1, col_v0, col_v1, val_v0, val_v1,
             rows_a, rows_b, rowsf_a, rowsf_b, zeros_v, acc_sh,
             sem_a, sem_b, semi_a, semi_b, sems_a, sems_b):
    c = lax.axis_index("c")
    s = lax.axis_index("s")
    tile = c * _NS + s

    def _zrow(i, _):
        for j in range(_D // 16):
            zeros_v[i, pl.ds(j * 16, 16)] = jnp.zeros((16,), jnp.float32)
        return 0
    lax.fori_loop(0, _ZBUF, _zrow, 0)
    for k in range(_ROWS_PER_TILE // _ZBUF):
        pltpu.sync_copy(zeros_v, acc_sh.at[pl.ds(s * _ROWS_PER_TILE + k * _ZBUF, _ZBUF)])

    @pl.when(s == 0)
    def _zero_rem():
        pltpu.sync_copy(zeros_v.at[pl.ds(0, _ROWS_REM)],
                        acc_sh.at[pl.ds(_NS * _ROWS_PER_TILE, _ROWS_REM)])
    plsc.subcore_barrier()

    e_base = tile * _E_PER_TILE
    bufs = ((row_v0, col_v0, val_v0, rows_a, rowsf_a, sem_a, semi_a, sems_a),
            (row_v1, col_v1, val_v1, rows_b, rowsf_b, sem_b, semi_b, sems_b))

    def _fetch_idx(b, g):
        rv, cv, vv, _, _, _, si, _ = bufs[b]
        base = e_base + g * _E_CHK
        pltpu.async_copy(row_hbm.at[pl.ds(base, _E_CHK)], rv, si)
        pltpu.async_copy(col_hbm.at[pl.ds(base, _E_CHK)], cv, si)
        pltpu.async_copy(val_hbm.at[pl.ds(base, _E_CHK)], vv, si)

    def _issue_gather(b):
        rv, cv, vv, rows, _, sg, si, _ = bufs[b]
        pltpu.make_async_copy(row_hbm.at[pl.ds(0, _E_CHK)], rv, si).wait()
        pltpu.make_async_copy(row_hbm.at[pl.ds(0, _E_CHK)], cv, si).wait()
        pltpu.make_async_copy(val_hbm.at[pl.ds(0, _E_CHK)], vv, si).wait()
        pltpu.async_copy(h_hbm.at[cv], rows, sg)

    def _wait_scatter(b):
        rv, cv, vv, rows, rf, sg, si, ss = bufs[b]
        pltpu.make_async_copy(rf, acc_sh.at[rv], ss).wait()

    def _drain(b):
        rv, cv, vv, rows, rf, sg, si, ss = bufs[b]
        pltpu.make_async_copy(h_hbm.at[cv], rows, sg).wait()

        def _edge(e2, _):
            for u in range(2):
                e = 2 * e2 + u
                vb = plsc.load_gather(vv, [jnp.full((16,), e, jnp.int32)])
                for j in range(_D // 32):
                    w = rows[e, pl.ds(j * 16, 16)]
                    lo = plsc.bitcast(w << 16, jnp.float32)
                    hi = plsc.bitcast(w & jnp.int32(-65536), jnp.float32)
                    rf[e, pl.ds(j * 32, 16)] = lo * vb
                    rf[e, pl.ds(j * 32 + 16, 16)] = hi * vb
            return 0
        lax.fori_loop(0, _E_CHK // 2, _edge, 0)
        pltpu.async_copy(rf, acc_sh.at[rv], ss, add=True)

    _fetch_idx(0, 0)
    _fetch_idx(1, 1)
    _issue_gather(0)

    def _pair(p, _):
        g0 = 2 * p
        more = p < _CHK_PER_TILE // 2 - 1
        _issue_gather(1)

        @pl.when(p > 0)
        def _ws0():
            _wait_scatter(0)
        _drain(0)

        @pl.when(more)
        def _pf0():
            _fetch_idx(0, g0 + 2)

        @pl.when(p > 0)
        def _ws1():
            _wait_scatter(1)
        _drain(1)

        @pl.when(more)
        def _pf1():
            _fetch_idx(1, g0 + 3)
            _issue_gather(0)
        return 0
    lax.fori_loop(0, _CHK_PER_TILE // 2, _pair, 0)
    _wait_scatter(0)
    _wait_scatter(1)

    plsc.subcore_barrier()

    pltpu.sync_copy(acc_sh.at[pl.ds(s * _ROWS_PER_TILE, _ROWS_PER_TILE)],
                    out_hbm.at[c, pl.ds(s * _ROWS_PER_TILE, _ROWS_PER_TILE)])

    @pl.when(s == 0)
    def _write_rem():
        pltpu.sync_copy(acc_sh.at[pl.ds(_NS * _ROWS_PER_TILE, _ROWS_REM)],
                        out_hbm.at[c, pl.ds(_NS * _ROWS_PER_TILE, _ROWS_REM)])


@jax.jit
def _sc_segment_sum(H, row, col, val):
    mesh = plsc.VectorSubcoreMesh(core_axis_name="c", subcore_axis_name="s")
    return pl.kernel(
        _sc_body,
        out_type=jax.ShapeDtypeStruct((_NC, _N_NODES, _D), jnp.float32),
        mesh=mesh,
        compiler_params=pltpu.CompilerParams(needs_layout_passes=False,
                                             use_tc_tiling_on_sc=False),
        scratch_types=[
            pltpu.VMEM((_E_CHK,), jnp.int32),
            pltpu.VMEM((_E_CHK,), jnp.int32),
            pltpu.VMEM((_E_CHK,), jnp.int32),
            pltpu.VMEM((_E_CHK,), jnp.int32),
            pltpu.VMEM((_E_CHK,), jnp.float32),
            pltpu.VMEM((_E_CHK,), jnp.float32),
            pltpu.VMEM((_E_CHK, _D // 2), jnp.int32),
            pltpu.VMEM((_E_CHK, _D // 2), jnp.int32),
            pltpu.VMEM((_E_CHK, _D), jnp.float32),
            pltpu.VMEM((_E_CHK, _D), jnp.float32),
            pltpu.VMEM((_ZBUF, _D), jnp.float32),
            pltpu.VMEM_SHARED((_N_NODES, _D), jnp.float32),
            pltpu.SemaphoreType.DMA,
            pltpu.SemaphoreType.DMA,
            pltpu.SemaphoreType.DMA,
            pltpu.SemaphoreType.DMA,
            pltpu.SemaphoreType.DMA,
            pltpu.SemaphoreType.DMA,
        ],
    )(H, row, col, val)


def _tc_body(h0_ref, h1_ref, scale_ref, shift_ref, w_ref, b_ref, o_ref):
    x = h0_ref[...] + h1_ref[...]
    x = x * scale_ref[...] + shift_ref[...]
    y = jnp.dot(x, w_ref[...], preferred_element_type=jnp.float32) + b_ref[...]
    o_ref[...] = 0.5 * y * (1.0 + lax.erf(y * 0.7071067811865476))


@jax.jit
def _tc_ffn(h0, h1, scale, shift, W, b):
    blk = 1000
    grid = (_N_NODES // blk,)
    return pl.pallas_call(
        _tc_body,
        grid=grid,
        in_specs=[
            pl.BlockSpec((blk, _D), lambda i: (i, 0)),
            pl.BlockSpec((blk, _D), lambda i: (i, 0)),
            pl.BlockSpec((1, _D), lambda i: (0, 0)),
            pl.BlockSpec((1, _D), lambda i: (0, 0)),
            pl.BlockSpec((_D, _D), lambda i: (0, 0)),
            pl.BlockSpec((1, _D), lambda i: (0, 0)),
        ],
        out_specs=pl.BlockSpec((blk, _D), lambda i: (i, 0)),
        out_shape=jax.ShapeDtypeStruct((_N_NODES, _D), jnp.float32),
    )(h0, h1, scale, shift, W, b)


def kernel(H, edge_index, adj_values, gamma, beta, moving_mean, moving_var, W, b):
    npad = _E_PAD - _N_EDGES
    row = jnp.pad(edge_index[0], (0, npad))
    col = jnp.pad(edge_index[1], (0, npad))
    val = jnp.pad(adj_values, (0, npad))
    perm = jnp.arange(_D).reshape(4, 2, 16).transpose(0, 2, 1).reshape(_D)
    hb = H[:, perm].astype(jnp.bfloat16).reshape(_N_NODES, _D // 2, 2)
    hpacked = lax.bitcast_convert_type(hb, jnp.int32)
    hpart = _sc_segment_sum(hpacked, row, col, val)
    scale = gamma * lax.rsqrt(moving_var + _BN_EPS)
    shift = beta - moving_mean * scale
    return _tc_ffn(hpart[0], hpart[1], scale.reshape(1, _D),
                   shift.reshape(1, _D), W, b.reshape(1, _D))
